# baseline (speedup 1.0000x reference)
import functools

import jax
import jax.numpy as jnp
from jax import lax
from jax.experimental import pallas as pl
from jax.experimental.pallas import tpu as pltpu
from jax.experimental.pallas import tpu_sc as plsc

_N = 10000
_E = 320000
_D = 128
_NG = 64
_NCLS = 10
_EPS = 0.0

_NSC = 2
_NTEC = 16
_NW = _NSC * _NTEC
_EPT = _E // _NW
_CH = 80
_NCHUNK = _EPT // _CH
_K = 3
_NFULL = (_NCHUNK // (2 * _K)) * 2
_NGRP2 = _NFULL // 2
_NTAIL = _NCHUNK - _NFULL * _K
_RPS = 632
_RPS_LAST = _N - (_NTEC - 1) * _RPS


def _sc_agg(x, src, dst, zrows):
    mesh = plsc.VectorSubcoreMesh(core_axis_name="c", subcore_axis_name="s")

    @functools.partial(
        pl.kernel,
        out_type=jax.ShapeDtypeStruct((_NSC, _N, _D), jnp.float32),
        mesh=mesh,
        scratch_types=[
            pltpu.VMEM((2 * _K, _CH), jnp.int32),
            pltpu.VMEM((2 * _K, _CH), jnp.int32),
            pltpu.VMEM((_K, _CH, _D), jnp.float32),
            pltpu.VMEM_SHARED((_N, _D), jnp.float32),
            pltpu.SemaphoreType.DMA((2,)),
            pltpu.SemaphoreType.DMA((_K,)),
            pltpu.SemaphoreType.DMA((_K,)),
        ],
    )
    def k(x_hbm, src_hbm, dst_hbm, z_hbm, out_hbm, src_v, dst_v, rows_v, agg_sh,
          isem, gsem, ssem):
        cid = lax.axis_index("c")
        sid = lax.axis_index("s")
        wid = sid * _NSC + cid

        rbase = sid * _RPS
        ebase = wid * _EPT

        def _idx_fetch(chunk, row, slot):
            off = ebase + chunk * _CH
            pltpu.async_copy(src_hbm.at[pl.ds(off, _CH)], src_v.at[row],
                             isem.at[slot])
            pltpu.async_copy(dst_hbm.at[pl.ds(off, _CH)], dst_v.at[row],
                             isem.at[slot])

        def _drain(src_ref, dst_ref, sem):
            pltpu.make_async_copy(src_ref, dst_ref, sem).wait()

        def _idx_drain(row, slot):
            _drain(src_hbm.at[pl.ds(0, _CH)], src_v.at[row], isem.at[slot])
            _drain(dst_hbm.at[pl.ds(0, _CH)], dst_v.at[row], isem.at[slot])

        for s in range(2):
            for b in range(_K):
                _idx_fetch(s * _K + b, s * _K + b, s)

        @pl.when(sid < _NTEC - 1)
        def _():
            pltpu.sync_copy(z_hbm, agg_sh.at[pl.ds(rbase, _RPS)])

        @pl.when(sid == _NTEC - 1)
        def _():
            pltpu.sync_copy(z_hbm.at[pl.ds(0, _RPS_LAST)],
                            agg_sh.at[pl.ds(rbase, _RPS_LAST)])

        plsc.subcore_barrier()

        def _gather(chunk, row, b):
            return pltpu.async_copy(x_hbm.at[src_v.at[row]], rows_v.at[b],
                                    gsem.at[b])

        def _scatter_add(row, b):
            pltpu.async_copy(rows_v.at[b], agg_sh.at[dst_v.at[row]],
                             ssem.at[b], add=True)

        @pl.loop(0, _NGRP2)
        def _(jj):
            for s in range(2):
                j = 2 * jj + s
                s1 = 1 - s
                for b in range(_K):
                    @pl.when(j > 0)
                    def _():
                        _drain(z_hbm.at[pl.ds(0, _CH)], rows_v.at[b],
                               ssem.at[b])

                    nc = (j + 1) * _K + b

                    @pl.when(nc < _NCHUNK)
                    def _():
                        _idx_fetch(nc, s1 * _K + b, s1)

                gathers = []
                for b in range(_K):
                    _idx_drain(s * _K + b, s)
                    gathers.append(_gather(j * _K + b, s * _K + b, b))
                for b in range(_K):
                    gathers[b].wait()
                    _scatter_add(s * _K + b, b)

        c0 = _NFULL * _K
        for e in range(_K, _NTAIL):
            c = c0 + e
            _idx_fetch(c, c % (2 * _K), (c % (2 * _K)) // _K)
        gathers = []
        for e in range(_NTAIL):
            c = c0 + e
            b = e % _K
            row = c % (2 * _K)
            if e >= _K:
                gathers[b].wait()
                _scatter_add((c - _K) % (2 * _K), b)
            _drain(z_hbm.at[pl.ds(0, _CH)], rows_v.at[b], ssem.at[b])
            _idx_drain(row, row // _K)
            g = _gather(c, row, b)
            if e < _K:
                gathers.append(g)
            else:
                gathers[b] = g
        for e in range(_NTAIL - _K, _NTAIL):
            c = c0 + e
            b = e % _K
            gathers[b].wait()
            _scatter_add(c % (2 * _K), b)
        for b in range(_K):
            _drain(z_hbm.at[pl.ds(0, _CH)], rows_v.at[b], ssem.at[b])

        plsc.subcore_barrier()

        @pl.when(sid < _NTEC - 1)
        def _():
            pltpu.sync_copy(agg_sh.at[pl.ds(rbase, _RPS)],
                            out_hbm.at[cid, pl.ds(rbase, _RPS)])

        @pl.when(sid == _NTEC - 1)
        def _():
            pltpu.sync_copy(agg_sh.at[pl.ds(rbase, _RPS_LAST)],
                            out_hbm.at[cid, pl.ds(rbase, _RPS_LAST)])

    return k(x, src, dst, zrows)


def _mlp_body(p_ref, x_ref, w1_ref, b1_ref, w2_ref, b2_ref, o_ref):
    agg = p_ref[0] + p_ref[1] + (1.0 + _EPS) * x_ref[...]
    h = jnp.dot(agg, w1_ref[...], preferred_element_type=jnp.float32) + b1_ref[...]
    h = jnp.maximum(h, 0.0)
    y = jnp.dot(h, w2_ref[...], preferred_element_type=jnp.float32) + b2_ref[...]
    o_ref[...] = jnp.maximum(y, 0.0)


def _mlp(partial, x, W1, b1, W2, b2):
    BN = 1000
    return pl.pallas_call(
        _mlp_body,
        grid=(_N // BN,),
        in_specs=[
            pl.BlockSpec((_NSC, BN, _D), lambda i: (0, i, 0)),
            pl.BlockSpec((BN, _D), lambda i: (i, 0)),
            pl.BlockSpec((_D, _D), lambda i: (0, 0)),
            pl.BlockSpec((1, _D), lambda i: (0, 0)),
            pl.BlockSpec((_D, _D), lambda i: (0, 0)),
            pl.BlockSpec((1, _D), lambda i: (0, 0)),
        ],
        out_specs=pl.BlockSpec((BN, _D), lambda i: (i, 0)),
        out_shape=jax.ShapeDtypeStruct((_N, _D), jnp.float32),
    )(partial, x, W1, b1, W2, b2)


def _head_body(x_ref, b_ref, w1_ref, b1_ref, w2_ref, b2_ref, o_ref, acc, cnt):
    i = pl.program_id(0)
    ng = pl.num_programs(0)

    @pl.when(i == 0)
    def _():
        acc[...] = jnp.zeros_like(acc)
        cnt[...] = jnp.zeros_like(cnt)

    bids = b_ref[0]
    gids = lax.broadcasted_iota(jnp.int32, (_NG, bids.shape[1]), 0)
    oh = (gids == bids).astype(jnp.float32)
    acc[...] += jnp.dot(oh, x_ref[...], preferred_element_type=jnp.float32)
    cnt[...] += jnp.sum(oh, axis=1, keepdims=True)

    @pl.when(i == ng - 1)
    def _():
        pooled = acc[...] / jnp.maximum(cnt[...], 1.0)
        h = jnp.dot(pooled, w1_ref[...], preferred_element_type=jnp.float32) + b1_ref[...]
        h = jnp.maximum(h, 0.0)
        logits = jnp.dot(h, w2_ref[...], preferred_element_type=jnp.float32) + b2_ref[...]
        m = jnp.max(logits, axis=-1, keepdims=True)
        lse = jnp.log(jnp.sum(jnp.exp(logits - m), axis=-1, keepdims=True)) + m
        o_ref[...] = logits - lse


def _head(xl, batch, lin1_W, lin1_b, lin2_W, lin2_b):
    BN = 1000
    G = _N // BN
    batch3 = batch.reshape(G, 1, BN)
    return pl.pallas_call(
        _head_body,
        grid=(G,),
        in_specs=[
            pl.BlockSpec((BN, _D), lambda i: (i, 0)),
            pl.BlockSpec((1, 1, BN), lambda i: (i, 0, 0)),
            pl.BlockSpec((_D, _D), lambda i: (0, 0)),
            pl.BlockSpec((1, _D), lambda i: (0, 0)),
            pl.BlockSpec((_D, _NCLS), lambda i: (0, 0)),
            pl.BlockSpec((1, _NCLS), lambda i: (0, 0)),
        ],
        out_specs=pl.BlockSpec((_NG, _NCLS), lambda i: (0, 0)),
        out_shape=jax.ShapeDtypeStruct((_NG, _NCLS), jnp.float32),
        scratch_shapes=[
            pltpu.VMEM((_NG, _D), jnp.float32),
            pltpu.VMEM((_NG, 1), jnp.float32),
        ],
    )(xl, batch3, lin1_W, lin1_b, lin2_W, lin2_b)


def kernel(x, edge_index, batch,
           conv0_W1, conv0_b1, conv0_W2, conv0_b2,
           conv1_W1, conv1_b1, conv1_W2, conv1_b2,
           conv2_W1, conv2_b1, conv2_W2, conv2_b2,
           lin1_W, lin1_b, lin2_W, lin2_b):
    src = edge_index[0]
    dst = edge_index[1]
    zrows = jnp.zeros((_RPS, _D), jnp.float32)

    h = x
    for (W1, b1, W2, b2) in (
        (conv0_W1, conv0_b1, conv0_W2, conv0_b2),
        (conv1_W1, conv1_b1, conv1_W2, conv1_b2),
        (conv2_W1, conv2_b1, conv2_W2, conv2_b2),
    ):
        partial = _sc_agg(h, src, dst, zrows)
        h = _mlp(partial, h, W1, b1.reshape(1, _D), W2, b2.reshape(1, _D))

    logsoft = _head(h, batch, lin1_W, lin1_b.reshape(1, _D),
                    lin2_W, lin2_b.reshape(1, _NCLS))
    return (logsoft, h)


# speedup vs baseline: 1.0250x; 1.0250x over previous
import functools

import jax
import jax.numpy as jnp
from jax import lax
from jax.experimental import pallas as pl
from jax.experimental.pallas import tpu as pltpu
from jax.experimental.pallas import tpu_sc as plsc

_N = 10000
_E = 320000
_D = 128
_NG = 64
_NCLS = 10
_EPS = 0.0

_NSC = 2
_NTEC = 16
_NW = _NSC * _NTEC
_EPT = _E // _NW
_CH = 80
_NCHUNK = _EPT // _CH
_K = 3
_NFULL = (_NCHUNK // (2 * _K)) * 2
_NGRP2 = _NFULL // 2
_NTAIL = _NCHUNK - _NFULL * _K
_RPS = 632
_RPS_LAST = _N - (_NTEC - 1) * _RPS


def _sc_agg(x, src, dst, zrows):
    mesh = plsc.VectorSubcoreMesh(core_axis_name="c", subcore_axis_name="s")

    @functools.partial(
        pl.kernel,
        out_type=jax.ShapeDtypeStruct((_NSC, _N, _D), jnp.float32),
        mesh=mesh,
        scratch_types=[
            pltpu.VMEM((2 * _K, _CH), jnp.int32),
            pltpu.VMEM((2 * _K, _CH), jnp.int32),
            pltpu.VMEM((_K, _CH, _D), jnp.float32),
            pltpu.VMEM_SHARED((_N, _D), jnp.float32),
            pltpu.SemaphoreType.DMA((2,)),
            pltpu.SemaphoreType.DMA((_K,)),
            pltpu.SemaphoreType.DMA((_K,)),
        ],
    )
    def k(x_hbm, src_hbm, dst_hbm, z_hbm, out_hbm, src_v, dst_v, rows_v, agg_sh,
          isem, gsem, ssem):
        cid = lax.axis_index("c")
        sid = lax.axis_index("s")
        wid = sid * _NSC + cid

        rbase = sid * _RPS
        ebase = wid * _EPT

        def _idx_fetch(chunk, row, slot):
            off = ebase + chunk * _CH
            pltpu.async_copy(src_hbm.at[pl.ds(off, _CH)], src_v.at[row],
                             isem.at[slot])
            pltpu.async_copy(dst_hbm.at[pl.ds(off, _CH)], dst_v.at[row],
                             isem.at[slot])

        def _drain(src_ref, dst_ref, sem):
            pltpu.make_async_copy(src_ref, dst_ref, sem).wait()

        def _idx_drain(row, slot):
            _drain(src_hbm.at[pl.ds(0, _CH)], src_v.at[row], isem.at[slot])
            _drain(dst_hbm.at[pl.ds(0, _CH)], dst_v.at[row], isem.at[slot])

        for s in range(2):
            for b in range(_K):
                _idx_fetch(s * _K + b, s * _K + b, s)

        for last in (False, True):
            n_own = _RPS_LAST if last else _RPS
            cond = (sid == _NTEC - 1) if last else (sid < _NTEC - 1)

            @pl.when(jnp.logical_and(cond, cid == 0))
            def _():
                pltpu.sync_copy(x_hbm.at[pl.ds(rbase, n_own)],
                                agg_sh.at[pl.ds(rbase, n_own)])

            @pl.when(jnp.logical_and(cond, cid == 1))
            def _():
                pltpu.sync_copy(z_hbm.at[pl.ds(rbase, n_own)],
                                agg_sh.at[pl.ds(rbase, n_own)])

        plsc.subcore_barrier()

        def _gather(chunk, row, b):
            return pltpu.async_copy(x_hbm.at[src_v.at[row]], rows_v.at[b],
                                    gsem.at[b])

        def _scatter_add(row, b):
            pltpu.async_copy(rows_v.at[b], agg_sh.at[dst_v.at[row]],
                             ssem.at[b], add=True)

        @pl.loop(0, _NGRP2)
        def _(jj):
            for s in range(2):
                j = 2 * jj + s
                s1 = 1 - s
                for b in range(_K):
                    @pl.when(j > 0)
                    def _():
                        _drain(z_hbm.at[pl.ds(0, _CH)], rows_v.at[b],
                               ssem.at[b])

                    nc = (j + 1) * _K + b

                    @pl.when(nc < _NCHUNK)
                    def _():
                        _idx_fetch(nc, s1 * _K + b, s1)

                gathers = []
                for b in range(_K):
                    _idx_drain(s * _K + b, s)
                    gathers.append(_gather(j * _K + b, s * _K + b, b))
                for b in range(_K):
                    gathers[b].wait()
                    _scatter_add(s * _K + b, b)

        c0 = _NFULL * _K
        for e in range(_K, _NTAIL):
            c = c0 + e
            _idx_fetch(c, c % (2 * _K), (c % (2 * _K)) // _K)
        gathers = []
        for e in range(_NTAIL):
            c = c0 + e
            b = e % _K
            row = c % (2 * _K)
            if e >= _K:
                gathers[b].wait()
                _scatter_add((c - _K) % (2 * _K), b)
            _drain(z_hbm.at[pl.ds(0, _CH)], rows_v.at[b], ssem.at[b])
            _idx_drain(row, row // _K)
            g = _gather(c, row, b)
            if e < _K:
                gathers.append(g)
            else:
                gathers[b] = g
        for e in range(_NTAIL - _K, _NTAIL):
            c = c0 + e
            b = e % _K
            gathers[b].wait()
            _scatter_add(c % (2 * _K), b)
        for b in range(_K):
            _drain(z_hbm.at[pl.ds(0, _CH)], rows_v.at[b], ssem.at[b])

        plsc.subcore_barrier()

        @pl.when(sid < _NTEC - 1)
        def _():
            pltpu.sync_copy(agg_sh.at[pl.ds(rbase, _RPS)],
                            out_hbm.at[cid, pl.ds(rbase, _RPS)])

        @pl.when(sid == _NTEC - 1)
        def _():
            pltpu.sync_copy(agg_sh.at[pl.ds(rbase, _RPS_LAST)],
                            out_hbm.at[cid, pl.ds(rbase, _RPS_LAST)])

    return k(x, src, dst, zrows)


def _mlp_body(p_ref, w1_ref, b1_ref, w2_ref, b2_ref, o_ref):
    agg = p_ref[0] + p_ref[1]
    h = jnp.dot(agg, w1_ref[...], preferred_element_type=jnp.float32) + b1_ref[...]
    h = jnp.maximum(h, 0.0)
    y = jnp.dot(h, w2_ref[...], preferred_element_type=jnp.float32) + b2_ref[...]
    o_ref[...] = jnp.maximum(y, 0.0)


def _mlp(partial, W1, b1, W2, b2):
    BN = 1000
    return pl.pallas_call(
        _mlp_body,
        grid=(_N // BN,),
        in_specs=[
            pl.BlockSpec((_NSC, BN, _D), lambda i: (0, i, 0)),
            pl.BlockSpec((_D, _D), lambda i: (0, 0)),
            pl.BlockSpec((1, _D), lambda i: (0, 0)),
            pl.BlockSpec((_D, _D), lambda i: (0, 0)),
            pl.BlockSpec((1, _D), lambda i: (0, 0)),
        ],
        out_specs=pl.BlockSpec((BN, _D), lambda i: (i, 0)),
        out_shape=jax.ShapeDtypeStruct((_N, _D), jnp.float32),
    )(partial, W1, b1, W2, b2)


def _mlp3_head_body(p_ref, w1_ref, b1_ref, w2_ref, b2_ref, bt_ref,
                    l1w_ref, l1b_ref, l2w_ref, l2b_ref,
                    y_ref, o_ref, acc, cnt):
    i = pl.program_id(0)
    ng = pl.num_programs(0)
    agg = p_ref[0] + p_ref[1]
    h = jnp.dot(agg, w1_ref[...], preferred_element_type=jnp.float32) + b1_ref[...]
    h = jnp.maximum(h, 0.0)
    y = jnp.dot(h, w2_ref[...], preferred_element_type=jnp.float32) + b2_ref[...]
    y = jnp.maximum(y, 0.0)
    y_ref[...] = y

    @pl.when(i == 0)
    def _():
        acc[...] = jnp.zeros_like(acc)
        cnt[...] = jnp.zeros_like(cnt)

    bids = bt_ref[0]
    gids = lax.broadcasted_iota(jnp.int32, (_NG, bids.shape[1]), 0)
    oh = (gids == bids).astype(jnp.float32)
    acc[...] += jnp.dot(oh, y, preferred_element_type=jnp.float32)
    cnt[...] += jnp.sum(oh, axis=1, keepdims=True)

    @pl.when(i == ng - 1)
    def _():
        pooled = acc[...] / jnp.maximum(cnt[...], 1.0)
        hh = jnp.dot(pooled, l1w_ref[...],
                     preferred_element_type=jnp.float32) + l1b_ref[...]
        hh = jnp.maximum(hh, 0.0)
        logits = jnp.dot(hh, l2w_ref[...],
                         preferred_element_type=jnp.float32) + l2b_ref[...]
        m = jnp.max(logits, axis=-1, keepdims=True)
        lse = jnp.log(jnp.sum(jnp.exp(logits - m), axis=-1, keepdims=True)) + m
        o_ref[...] = logits - lse


def _mlp3_head(partial, W1, b1, W2, b2, batch, lin1_W, lin1_b, lin2_W, lin2_b):
    BN = 1000
    G = _N // BN
    batch3 = batch.reshape(G, 1, BN)
    return pl.pallas_call(
        _mlp3_head_body,
        grid=(G,),
        in_specs=[
            pl.BlockSpec((_NSC, BN, _D), lambda i: (0, i, 0)),
            pl.BlockSpec((_D, _D), lambda i: (0, 0)),
            pl.BlockSpec((1, _D), lambda i: (0, 0)),
            pl.BlockSpec((_D, _D), lambda i: (0, 0)),
            pl.BlockSpec((1, _D), lambda i: (0, 0)),
            pl.BlockSpec((1, 1, BN), lambda i: (i, 0, 0)),
            pl.BlockSpec((_D, _D), lambda i: (0, 0)),
            pl.BlockSpec((1, _D), lambda i: (0, 0)),
            pl.BlockSpec((_D, _NCLS), lambda i: (0, 0)),
            pl.BlockSpec((1, _NCLS), lambda i: (0, 0)),
        ],
        out_specs=[
            pl.BlockSpec((BN, _D), lambda i: (i, 0)),
            pl.BlockSpec((_NG, _NCLS), lambda i: (0, 0)),
        ],
        out_shape=[
            jax.ShapeDtypeStruct((_N, _D), jnp.float32),
            jax.ShapeDtypeStruct((_NG, _NCLS), jnp.float32),
        ],
        scratch_shapes=[
            pltpu.VMEM((_NG, _D), jnp.float32),
            pltpu.VMEM((_NG, 1), jnp.float32),
        ],
    )(partial, W1, b1, W2, b2, batch3, lin1_W, lin1_b, lin2_W, lin2_b)


def _head_body(x_ref, b_ref, w1_ref, b1_ref, w2_ref, b2_ref, o_ref, acc, cnt):
    i = pl.program_id(0)
    ng = pl.num_programs(0)

    @pl.when(i == 0)
    def _():
        acc[...] = jnp.zeros_like(acc)
        cnt[...] = jnp.zeros_like(cnt)

    bids = b_ref[0]
    gids = lax.broadcasted_iota(jnp.int32, (_NG, bids.shape[1]), 0)
    oh = (gids == bids).astype(jnp.float32)
    acc[...] += jnp.dot(oh, x_ref[...], preferred_element_type=jnp.float32)
    cnt[...] += jnp.sum(oh, axis=1, keepdims=True)

    @pl.when(i == ng - 1)
    def _():
        pooled = acc[...] / jnp.maximum(cnt[...], 1.0)
        h = jnp.dot(pooled, w1_ref[...], preferred_element_type=jnp.float32) + b1_ref[...]
        h = jnp.maximum(h, 0.0)
        logits = jnp.dot(h, w2_ref[...], preferred_element_type=jnp.float32) + b2_ref[...]
        m = jnp.max(logits, axis=-1, keepdims=True)
        lse = jnp.log(jnp.sum(jnp.exp(logits - m), axis=-1, keepdims=True)) + m
        o_ref[...] = logits - lse


def _head(xl, batch, lin1_W, lin1_b, lin2_W, lin2_b):
    BN = 1000
    G = _N // BN
    batch3 = batch.reshape(G, 1, BN)
    return pl.pallas_call(
        _head_body,
        grid=(G,),
        in_specs=[
            pl.BlockSpec((BN, _D), lambda i: (i, 0)),
            pl.BlockSpec((1, 1, BN), lambda i: (i, 0, 0)),
            pl.BlockSpec((_D, _D), lambda i: (0, 0)),
            pl.BlockSpec((1, _D), lambda i: (0, 0)),
            pl.BlockSpec((_D, _NCLS), lambda i: (0, 0)),
            pl.BlockSpec((1, _NCLS), lambda i: (0, 0)),
        ],
        out_specs=pl.BlockSpec((_NG, _NCLS), lambda i: (0, 0)),
        out_shape=jax.ShapeDtypeStruct((_NG, _NCLS), jnp.float32),
        scratch_shapes=[
            pltpu.VMEM((_NG, _D), jnp.float32),
            pltpu.VMEM((_NG, 1), jnp.float32),
        ],
    )(xl, batch3, lin1_W, lin1_b, lin2_W, lin2_b)


def kernel(x, edge_index, batch,
           conv0_W1, conv0_b1, conv0_W2, conv0_b2,
           conv1_W1, conv1_b1, conv1_W2, conv1_b2,
           conv2_W1, conv2_b1, conv2_W2, conv2_b2,
           lin1_W, lin1_b, lin2_W, lin2_b):
    src = edge_index[0]
    dst = edge_index[1]
    zfull = jnp.zeros((_N, _D), jnp.float32)

    h = x
    for (W1, b1, W2, b2) in (
        (conv0_W1, conv0_b1, conv0_W2, conv0_b2),
        (conv1_W1, conv1_b1, conv1_W2, conv1_b2),
    ):
        partial = _sc_agg(h, src, dst, zfull)
        h = _mlp(partial, W1, b1.reshape(1, _D), W2, b2.reshape(1, _D))

    partial = _sc_agg(h, src, dst, zfull)
    h, logsoft = _mlp3_head(partial, conv2_W1, conv2_b1.reshape(1, _D),
                            conv2_W2, conv2_b2.reshape(1, _D),
                            batch, lin1_W, lin1_b.reshape(1, _D),
                            lin2_W, lin2_b.reshape(1, _NCLS))
    return (logsoft, h)


# speedup vs baseline: 1.0690x; 1.0429x over previous
import functools

import jax
import jax.numpy as jnp
from jax import lax
from jax.experimental import pallas as pl
from jax.experimental.pallas import tpu as pltpu
from jax.experimental.pallas import tpu_sc as plsc

_N = 10000
_E = 320000
_D = 128
_NG = 64
_NCLS = 10
_EPS = 0.0

_NSC = 2
_NTEC = 16
_NW = _NSC * _NTEC
_EPT = _E // _NW
_CH = 40
_NCHUNK = _EPT // _CH
_K = 6
_NFULL = (_NCHUNK // (2 * _K)) * 2
_NGRP2 = _NFULL // 2
_NTAIL = _NCHUNK - _NFULL * _K
_RPS = 632
_RPS_LAST = _N - (_NTEC - 1) * _RPS


def _sc_agg(x, src, dst, zrows):
    mesh = plsc.VectorSubcoreMesh(core_axis_name="c", subcore_axis_name="s")

    @functools.partial(
        pl.kernel,
        out_type=jax.ShapeDtypeStruct((_NSC, _N, _D), jnp.float32),
        mesh=mesh,
        scratch_types=[
            pltpu.VMEM((2 * _K, _CH), jnp.int32),
            pltpu.VMEM((2 * _K, _CH), jnp.int32),
            pltpu.VMEM((_K, _CH, _D), jnp.float32),
            pltpu.VMEM_SHARED((_N, _D), jnp.float32),
            pltpu.SemaphoreType.DMA((2,)),
            pltpu.SemaphoreType.DMA((_K,)),
            pltpu.SemaphoreType.DMA((_K,)),
        ],
    )
    def k(x_hbm, src_hbm, dst_hbm, z_hbm, out_hbm, src_v, dst_v, rows_v, agg_sh,
          isem, gsem, ssem):
        cid = lax.axis_index("c")
        sid = lax.axis_index("s")
        wid = sid * _NSC + cid

        rbase = sid * _RPS
        ebase = wid * _EPT

        def _idx_fetch(chunk, row, slot):
            off = ebase + chunk * _CH
            pltpu.async_copy(src_hbm.at[pl.ds(off, _CH)], src_v.at[row],
                             isem.at[slot])
            pltpu.async_copy(dst_hbm.at[pl.ds(off, _CH)], dst_v.at[row],
                             isem.at[slot])

        def _drain(src_ref, dst_ref, sem):
            pltpu.make_async_copy(src_ref, dst_ref, sem).wait()

        def _idx_drain(row, slot):
            _drain(src_hbm.at[pl.ds(0, _CH)], src_v.at[row], isem.at[slot])
            _drain(dst_hbm.at[pl.ds(0, _CH)], dst_v.at[row], isem.at[slot])

        for s in range(2):
            for b in range(_K):
                _idx_fetch(s * _K + b, s * _K + b, s)

        for last in (False, True):
            n_own = _RPS_LAST if last else _RPS
            cond = (sid == _NTEC - 1) if last else (sid < _NTEC - 1)

            @pl.when(jnp.logical_and(cond, cid == 0))
            def _():
                pltpu.sync_copy(x_hbm.at[pl.ds(rbase, n_own)],
                                agg_sh.at[pl.ds(rbase, n_own)])

            @pl.when(jnp.logical_and(cond, cid == 1))
            def _():
                pltpu.sync_copy(z_hbm.at[pl.ds(rbase, n_own)],
                                agg_sh.at[pl.ds(rbase, n_own)])

        plsc.subcore_barrier()

        def _gather(chunk, row, b):
            return pltpu.async_copy(x_hbm.at[src_v.at[row]], rows_v.at[b],
                                    gsem.at[b])

        def _scatter_add(row, b):
            pltpu.async_copy(rows_v.at[b], agg_sh.at[dst_v.at[row]],
                             ssem.at[b], add=True)

        @pl.loop(0, _NGRP2)
        def _(jj):
            for s in range(2):
                j = 2 * jj + s
                s1 = 1 - s
                for b in range(_K):
                    @pl.when(j > 0)
                    def _():
                        _drain(z_hbm.at[pl.ds(0, _CH)], rows_v.at[b],
                               ssem.at[b])

                    nc = (j + 1) * _K + b

                    @pl.when(nc < _NCHUNK)
                    def _():
                        _idx_fetch(nc, s1 * _K + b, s1)

                gathers = []
                for b in range(_K):
                    _idx_drain(s * _K + b, s)
                    gathers.append(_gather(j * _K + b, s * _K + b, b))
                for b in range(_K):
                    gathers[b].wait()
                    _scatter_add(s * _K + b, b)

        c0 = _NFULL * _K
        for e in range(_K, _NTAIL):
            c = c0 + e
            _idx_fetch(c, c % (2 * _K), (c % (2 * _K)) // _K)
        gathers = []
        for e in range(_NTAIL):
            c = c0 + e
            b = e % _K
            row = c % (2 * _K)
            if e >= _K:
                gathers[b].wait()
                _scatter_add((c - _K) % (2 * _K), b)
            _drain(z_hbm.at[pl.ds(0, _CH)], rows_v.at[b], ssem.at[b])
            _idx_drain(row, row // _K)
            g = _gather(c, row, b)
            if e < _K:
                gathers.append(g)
            else:
                gathers[b] = g
        for e in range(_NTAIL - _K, _NTAIL):
            c = c0 + e
            b = e % _K
            gathers[b].wait()
            _scatter_add(c % (2 * _K), b)
        for b in range(_K):
            _drain(z_hbm.at[pl.ds(0, _CH)], rows_v.at[b], ssem.at[b])

        plsc.subcore_barrier()

        @pl.when(sid < _NTEC - 1)
        def _():
            pltpu.sync_copy(agg_sh.at[pl.ds(rbase, _RPS)],
                            out_hbm.at[cid, pl.ds(rbase, _RPS)])

        @pl.when(sid == _NTEC - 1)
        def _():
            pltpu.sync_copy(agg_sh.at[pl.ds(rbase, _RPS_LAST)],
                            out_hbm.at[cid, pl.ds(rbase, _RPS_LAST)])

    return k(x, src, dst, zrows)


def _mlp_body(p_ref, w1_ref, b1_ref, w2_ref, b2_ref, o_ref):
    agg = p_ref[0] + p_ref[1]
    h = jnp.dot(agg, w1_ref[...], preferred_element_type=jnp.float32) + b1_ref[...]
    h = jnp.maximum(h, 0.0)
    y = jnp.dot(h, w2_ref[...], preferred_element_type=jnp.float32) + b2_ref[...]
    o_ref[...] = jnp.maximum(y, 0.0)


def _mlp(partial, W1, b1, W2, b2):
    BN = 1000
    return pl.pallas_call(
        _mlp_body,
        grid=(_N // BN,),
        in_specs=[
            pl.BlockSpec((_NSC, BN, _D), lambda i: (0, i, 0)),
            pl.BlockSpec((_D, _D), lambda i: (0, 0)),
            pl.BlockSpec((1, _D), lambda i: (0, 0)),
            pl.BlockSpec((_D, _D), lambda i: (0, 0)),
            pl.BlockSpec((1, _D), lambda i: (0, 0)),
        ],
        out_specs=pl.BlockSpec((BN, _D), lambda i: (i, 0)),
        out_shape=jax.ShapeDtypeStruct((_N, _D), jnp.float32),
    )(partial, W1, b1, W2, b2)


def _mlp3_head_body(p_ref, w1_ref, b1_ref, w2_ref, b2_ref, bt_ref,
                    l1w_ref, l1b_ref, l2w_ref, l2b_ref,
                    y_ref, o_ref, acc, cnt):
    i = pl.program_id(0)
    ng = pl.num_programs(0)
    agg = p_ref[0] + p_ref[1]
    h = jnp.dot(agg, w1_ref[...], preferred_element_type=jnp.float32) + b1_ref[...]
    h = jnp.maximum(h, 0.0)
    y = jnp.dot(h, w2_ref[...], preferred_element_type=jnp.float32) + b2_ref[...]
    y = jnp.maximum(y, 0.0)
    y_ref[...] = y

    @pl.when(i == 0)
    def _():
        acc[...] = jnp.zeros_like(acc)
        cnt[...] = jnp.zeros_like(cnt)

    bids = bt_ref[0]
    gids = lax.broadcasted_iota(jnp.int32, (_NG, bids.shape[1]), 0)
    oh = (gids == bids).astype(jnp.float32)
    acc[...] += jnp.dot(oh, y, preferred_element_type=jnp.float32)
    cnt[...] += jnp.sum(oh, axis=1, keepdims=True)

    @pl.when(i == ng - 1)
    def _():
        pooled = acc[...] / jnp.maximum(cnt[...], 1.0)
        hh = jnp.dot(pooled, l1w_ref[...],
                     preferred_element_type=jnp.float32) + l1b_ref[...]
        hh = jnp.maximum(hh, 0.0)
        logits = jnp.dot(hh, l2w_ref[...],
                         preferred_element_type=jnp.float32) + l2b_ref[...]
        m = jnp.max(logits, axis=-1, keepdims=True)
        lse = jnp.log(jnp.sum(jnp.exp(logits - m), axis=-1, keepdims=True)) + m
        o_ref[...] = logits - lse


def _mlp3_head(partial, W1, b1, W2, b2, batch, lin1_W, lin1_b, lin2_W, lin2_b):
    BN = 1000
    G = _N // BN
    batch3 = batch.reshape(G, 1, BN)
    return pl.pallas_call(
        _mlp3_head_body,
        grid=(G,),
        in_specs=[
            pl.BlockSpec((_NSC, BN, _D), lambda i: (0, i, 0)),
            pl.BlockSpec((_D, _D), lambda i: (0, 0)),
            pl.BlockSpec((1, _D), lambda i: (0, 0)),
            pl.BlockSpec((_D, _D), lambda i: (0, 0)),
            pl.BlockSpec((1, _D), lambda i: (0, 0)),
            pl.BlockSpec((1, 1, BN), lambda i: (i, 0, 0)),
            pl.BlockSpec((_D, _D), lambda i: (0, 0)),
            pl.BlockSpec((1, _D), lambda i: (0, 0)),
            pl.BlockSpec((_D, _NCLS), lambda i: (0, 0)),
            pl.BlockSpec((1, _NCLS), lambda i: (0, 0)),
        ],
        out_specs=[
            pl.BlockSpec((BN, _D), lambda i: (i, 0)),
            pl.BlockSpec((_NG, _NCLS), lambda i: (0, 0)),
        ],
        out_shape=[
            jax.ShapeDtypeStruct((_N, _D), jnp.float32),
            jax.ShapeDtypeStruct((_NG, _NCLS), jnp.float32),
        ],
        scratch_shapes=[
            pltpu.VMEM((_NG, _D), jnp.float32),
            pltpu.VMEM((_NG, 1), jnp.float32),
        ],
    )(partial, W1, b1, W2, b2, batch3, lin1_W, lin1_b, lin2_W, lin2_b)


def _head_body(x_ref, b_ref, w1_ref, b1_ref, w2_ref, b2_ref, o_ref, acc, cnt):
    i = pl.program_id(0)
    ng = pl.num_programs(0)

    @pl.when(i == 0)
    def _():
        acc[...] = jnp.zeros_like(acc)
        cnt[...] = jnp.zeros_like(cnt)

    bids = b_ref[0]
    gids = lax.broadcasted_iota(jnp.int32, (_NG, bids.shape[1]), 0)
    oh = (gids == bids).astype(jnp.float32)
    acc[...] += jnp.dot(oh, x_ref[...], preferred_element_type=jnp.float32)
    cnt[...] += jnp.sum(oh, axis=1, keepdims=True)

    @pl.when(i == ng - 1)
    def _():
        pooled = acc[...] / jnp.maximum(cnt[...], 1.0)
        h = jnp.dot(pooled, w1_ref[...], preferred_element_type=jnp.float32) + b1_ref[...]
        h = jnp.maximum(h, 0.0)
        logits = jnp.dot(h, w2_ref[...], preferred_element_type=jnp.float32) + b2_ref[...]
        m = jnp.max(logits, axis=-1, keepdims=True)
        lse = jnp.log(jnp.sum(jnp.exp(logits - m), axis=-1, keepdims=True)) + m
        o_ref[...] = logits - lse


def _head(xl, batch, lin1_W, lin1_b, lin2_W, lin2_b):
    BN = 1000
    G = _N // BN
    batch3 = batch.reshape(G, 1, BN)
    return pl.pallas_call(
        _head_body,
        grid=(G,),
        in_specs=[
            pl.BlockSpec((BN, _D), lambda i: (i, 0)),
            pl.BlockSpec((1, 1, BN), lambda i: (i, 0, 0)),
            pl.BlockSpec((_D, _D), lambda i: (0, 0)),
            pl.BlockSpec((1, _D), lambda i: (0, 0)),
            pl.BlockSpec((_D, _NCLS), lambda i: (0, 0)),
            pl.BlockSpec((1, _NCLS), lambda i: (0, 0)),
        ],
        out_specs=pl.BlockSpec((_NG, _NCLS), lambda i: (0, 0)),
        out_shape=jax.ShapeDtypeStruct((_NG, _NCLS), jnp.float32),
        scratch_shapes=[
            pltpu.VMEM((_NG, _D), jnp.float32),
            pltpu.VMEM((_NG, 1), jnp.float32),
        ],
    )(xl, batch3, lin1_W, lin1_b, lin2_W, lin2_b)


def kernel(x, edge_index, batch,
           conv0_W1, conv0_b1, conv0_W2, conv0_b2,
           conv1_W1, conv1_b1, conv1_W2, conv1_b2,
           conv2_W1, conv2_b1, conv2_W2, conv2_b2,
           lin1_W, lin1_b, lin2_W, lin2_b):
    src = edge_index[0]
    dst = edge_index[1]
    zfull = jnp.zeros((_N, _D), jnp.float32)

    h = x
    for (W1, b1, W2, b2) in (
        (conv0_W1, conv0_b1, conv0_W2, conv0_b2),
        (conv1_W1, conv1_b1, conv1_W2, conv1_b2),
    ):
        partial = _sc_agg(h, src, dst, zfull)
        h = _mlp(partial, W1, b1.reshape(1, _D), W2, b2.reshape(1, _D))

    partial = _sc_agg(h, src, dst, zfull)
    h, logsoft = _mlp3_head(partial, conv2_W1, conv2_b1.reshape(1, _D),
                            conv2_W2, conv2_b2.reshape(1, _D),
                            batch, lin1_W, lin1_b.reshape(1, _D),
                            lin2_W, lin2_b.reshape(1, _NCLS))
    return (logsoft, h)


# speedup vs baseline: 1.0992x; 1.0283x over previous
import functools

import jax
import jax.numpy as jnp
from jax import lax
from jax.experimental import pallas as pl
from jax.experimental.pallas import tpu as pltpu
from jax.experimental.pallas import tpu_sc as plsc

_N = 10000
_E = 320000
_D = 128
_NG = 64
_NCLS = 10
_EPS = 0.0

_NSC = 2
_NTEC = 16
_NW = _NSC * _NTEC
_EPT = _E // _NW
_CH = 40
_NCHUNK = _EPT // _CH
_K = 7
_NFULL = (_NCHUNK // (2 * _K)) * 2
_NGRP2 = _NFULL // 2
_NTAIL = _NCHUNK - _NFULL * _K
_RPS = 632
_RPS_LAST = _N - (_NTEC - 1) * _RPS


def _sc_agg(x, src, dst, zrows):
    mesh = plsc.VectorSubcoreMesh(core_axis_name="c", subcore_axis_name="s")

    @functools.partial(
        pl.kernel,
        out_type=jax.ShapeDtypeStruct((_NSC, _N, _D), jnp.float32),
        mesh=mesh,
        scratch_types=[
            pltpu.VMEM((2 * _K, _CH), jnp.int32),
            pltpu.VMEM((2 * _K, _CH), jnp.int32),
            pltpu.VMEM((_K, _CH, _D), jnp.float32),
            pltpu.VMEM_SHARED((_N, _D), jnp.float32),
            pltpu.SemaphoreType.DMA((2,)),
            pltpu.SemaphoreType.DMA((_K,)),
            pltpu.SemaphoreType.DMA((_K,)),
        ],
    )
    def k(x_hbm, src_hbm, dst_hbm, z_hbm, out_hbm, src_v, dst_v, rows_v, agg_sh,
          isem, gsem, ssem):
        cid = lax.axis_index("c")
        sid = lax.axis_index("s")
        wid = sid * _NSC + cid

        rbase = sid * _RPS
        ebase = wid * _EPT

        def _idx_fetch(chunk, row, slot):
            off = ebase + chunk * _CH
            pltpu.async_copy(src_hbm.at[pl.ds(off, _CH)], src_v.at[row],
                             isem.at[slot])
            pltpu.async_copy(dst_hbm.at[pl.ds(off, _CH)], dst_v.at[row],
                             isem.at[slot])

        def _drain(src_ref, dst_ref, sem):
            pltpu.make_async_copy(src_ref, dst_ref, sem).wait()

        def _idx_drain(row, slot):
            _drain(src_hbm.at[pl.ds(0, _CH)], src_v.at[row], isem.at[slot])
            _drain(dst_hbm.at[pl.ds(0, _CH)], dst_v.at[row], isem.at[slot])

        for s in range(2):
            for b in range(_K):
                _idx_fetch(s * _K + b, s * _K + b, s)

        for last in (False, True):
            n_own = _RPS_LAST if last else _RPS
            cond = (sid == _NTEC - 1) if last else (sid < _NTEC - 1)

            @pl.when(jnp.logical_and(cond, cid == 0))
            def _():
                pltpu.sync_copy(x_hbm.at[pl.ds(rbase, n_own)],
                                agg_sh.at[pl.ds(rbase, n_own)])

            @pl.when(jnp.logical_and(cond, cid == 1))
            def _():
                pltpu.sync_copy(z_hbm.at[pl.ds(rbase, n_own)],
                                agg_sh.at[pl.ds(rbase, n_own)])

        plsc.subcore_barrier()

        def _gather(chunk, row, b):
            return pltpu.async_copy(x_hbm.at[src_v.at[row]], rows_v.at[b],
                                    gsem.at[b])

        def _scatter_add(row, b):
            pltpu.async_copy(rows_v.at[b], agg_sh.at[dst_v.at[row]],
                             ssem.at[b], add=True)

        @pl.loop(0, _NGRP2)
        def _(jj):
            for s in range(2):
                j = 2 * jj + s
                s1 = 1 - s
                for b in range(_K):
                    @pl.when(j > 0)
                    def _():
                        _drain(z_hbm.at[pl.ds(0, _CH)], rows_v.at[b],
                               ssem.at[b])

                    nc = (j + 1) * _K + b

                    @pl.when(nc < _NCHUNK)
                    def _():
                        _idx_fetch(nc, s1 * _K + b, s1)

                gathers = []
                for b in range(_K):
                    _idx_drain(s * _K + b, s)
                    gathers.append(_gather(j * _K + b, s * _K + b, b))
                for b in range(_K):
                    gathers[b].wait()
                    _scatter_add(s * _K + b, b)

        c0 = _NFULL * _K
        for e in range(_K, _NTAIL):
            c = c0 + e
            _idx_fetch(c, c % (2 * _K), (c % (2 * _K)) // _K)
        gathers = []
        for e in range(_NTAIL):
            c = c0 + e
            b = e % _K
            row = c % (2 * _K)
            if e >= _K:
                gathers[b].wait()
                _scatter_add((c - _K) % (2 * _K), b)
            _drain(z_hbm.at[pl.ds(0, _CH)], rows_v.at[b], ssem.at[b])
            _idx_drain(row, row // _K)
            g = _gather(c, row, b)
            if e < _K:
                gathers.append(g)
            else:
                gathers[b] = g
        for e in range(_NTAIL - _K, _NTAIL):
            c = c0 + e
            b = e % _K
            gathers[b].wait()
            _scatter_add(c % (2 * _K), b)
        for b in range(_K):
            _drain(z_hbm.at[pl.ds(0, _CH)], rows_v.at[b], ssem.at[b])

        plsc.subcore_barrier()

        @pl.when(sid < _NTEC - 1)
        def _():
            pltpu.sync_copy(agg_sh.at[pl.ds(rbase, _RPS)],
                            out_hbm.at[cid, pl.ds(rbase, _RPS)])

        @pl.when(sid == _NTEC - 1)
        def _():
            pltpu.sync_copy(agg_sh.at[pl.ds(rbase, _RPS_LAST)],
                            out_hbm.at[cid, pl.ds(rbase, _RPS_LAST)])

    return k(x, src, dst, zrows)


def _mlp_body(p_ref, w1_ref, b1_ref, w2_ref, b2_ref, o_ref):
    agg = p_ref[0] + p_ref[1]
    h = jnp.dot(agg, w1_ref[...], preferred_element_type=jnp.float32) + b1_ref[...]
    h = jnp.maximum(h, 0.0)
    y = jnp.dot(h, w2_ref[...], preferred_element_type=jnp.float32) + b2_ref[...]
    o_ref[...] = jnp.maximum(y, 0.0)


def _mlp(partial, W1, b1, W2, b2):
    BN = 1000
    return pl.pallas_call(
        _mlp_body,
        grid=(_N // BN,),
        in_specs=[
            pl.BlockSpec((_NSC, BN, _D), lambda i: (0, i, 0)),
            pl.BlockSpec((_D, _D), lambda i: (0, 0)),
            pl.BlockSpec((1, _D), lambda i: (0, 0)),
            pl.BlockSpec((_D, _D), lambda i: (0, 0)),
            pl.BlockSpec((1, _D), lambda i: (0, 0)),
        ],
        out_specs=pl.BlockSpec((BN, _D), lambda i: (i, 0)),
        out_shape=jax.ShapeDtypeStruct((_N, _D), jnp.float32),
    )(partial, W1, b1, W2, b2)


def _mlp3_head_body(p_ref, w1_ref, b1_ref, w2_ref, b2_ref, bt_ref,
                    l1w_ref, l1b_ref, l2w_ref, l2b_ref,
                    y_ref, o_ref, acc, cnt):
    i = pl.program_id(0)
    ng = pl.num_programs(0)
    agg = p_ref[0] + p_ref[1]
    h = jnp.dot(agg, w1_ref[...], preferred_element_type=jnp.float32) + b1_ref[...]
    h = jnp.maximum(h, 0.0)
    y = jnp.dot(h, w2_ref[...], preferred_element_type=jnp.float32) + b2_ref[...]
    y = jnp.maximum(y, 0.0)
    y_ref[...] = y

    @pl.when(i == 0)
    def _():
        acc[...] = jnp.zeros_like(acc)
        cnt[...] = jnp.zeros_like(cnt)

    bids = bt_ref[0]
    gids = lax.broadcasted_iota(jnp.int32, (_NG, bids.shape[1]), 0)
    oh = (gids == bids).astype(jnp.float32)
    acc[...] += jnp.dot(oh, y, preferred_element_type=jnp.float32)
    cnt[...] += jnp.sum(oh, axis=1, keepdims=True)

    @pl.when(i == ng - 1)
    def _():
        pooled = acc[...] / jnp.maximum(cnt[...], 1.0)
        hh = jnp.dot(pooled, l1w_ref[...],
                     preferred_element_type=jnp.float32) + l1b_ref[...]
        hh = jnp.maximum(hh, 0.0)
        logits = jnp.dot(hh, l2w_ref[...],
                         preferred_element_type=jnp.float32) + l2b_ref[...]
        m = jnp.max(logits, axis=-1, keepdims=True)
        lse = jnp.log(jnp.sum(jnp.exp(logits - m), axis=-1, keepdims=True)) + m
        o_ref[...] = logits - lse


def _mlp3_head(partial, W1, b1, W2, b2, batch, lin1_W, lin1_b, lin2_W, lin2_b):
    BN = 1000
    G = _N // BN
    batch3 = batch.reshape(G, 1, BN)
    return pl.pallas_call(
        _mlp3_head_body,
        grid=(G,),
        in_specs=[
            pl.BlockSpec((_NSC, BN, _D), lambda i: (0, i, 0)),
            pl.BlockSpec((_D, _D), lambda i: (0, 0)),
            pl.BlockSpec((1, _D), lambda i: (0, 0)),
            pl.BlockSpec((_D, _D), lambda i: (0, 0)),
            pl.BlockSpec((1, _D), lambda i: (0, 0)),
            pl.BlockSpec((1, 1, BN), lambda i: (i, 0, 0)),
            pl.BlockSpec((_D, _D), lambda i: (0, 0)),
            pl.BlockSpec((1, _D), lambda i: (0, 0)),
            pl.BlockSpec((_D, _NCLS), lambda i: (0, 0)),
            pl.BlockSpec((1, _NCLS), lambda i: (0, 0)),
        ],
        out_specs=[
            pl.BlockSpec((BN, _D), lambda i: (i, 0)),
            pl.BlockSpec((_NG, _NCLS), lambda i: (0, 0)),
        ],
        out_shape=[
            jax.ShapeDtypeStruct((_N, _D), jnp.float32),
            jax.ShapeDtypeStruct((_NG, _NCLS), jnp.float32),
        ],
        scratch_shapes=[
            pltpu.VMEM((_NG, _D), jnp.float32),
            pltpu.VMEM((_NG, 1), jnp.float32),
        ],
    )(partial, W1, b1, W2, b2, batch3, lin1_W, lin1_b, lin2_W, lin2_b)


def _head_body(x_ref, b_ref, w1_ref, b1_ref, w2_ref, b2_ref, o_ref, acc, cnt):
    i = pl.program_id(0)
    ng = pl.num_programs(0)

    @pl.when(i == 0)
    def _():
        acc[...] = jnp.zeros_like(acc)
        cnt[...] = jnp.zeros_like(cnt)

    bids = b_ref[0]
    gids = lax.broadcasted_iota(jnp.int32, (_NG, bids.shape[1]), 0)
    oh = (gids == bids).astype(jnp.float32)
    acc[...] += jnp.dot(oh, x_ref[...], preferred_element_type=jnp.float32)
    cnt[...] += jnp.sum(oh, axis=1, keepdims=True)

    @pl.when(i == ng - 1)
    def _():
        pooled = acc[...] / jnp.maximum(cnt[...], 1.0)
        h = jnp.dot(pooled, w1_ref[...], preferred_element_type=jnp.float32) + b1_ref[...]
        h = jnp.maximum(h, 0.0)
        logits = jnp.dot(h, w2_ref[...], preferred_element_type=jnp.float32) + b2_ref[...]
        m = jnp.max(logits, axis=-1, keepdims=True)
        lse = jnp.log(jnp.sum(jnp.exp(logits - m), axis=-1, keepdims=True)) + m
        o_ref[...] = logits - lse


def _head(xl, batch, lin1_W, lin1_b, lin2_W, lin2_b):
    BN = 1000
    G = _N // BN
    batch3 = batch.reshape(G, 1, BN)
    return pl.pallas_call(
        _head_body,
        grid=(G,),
        in_specs=[
            pl.BlockSpec((BN, _D), lambda i: (i, 0)),
            pl.BlockSpec((1, 1, BN), lambda i: (i, 0, 0)),
            pl.BlockSpec((_D, _D), lambda i: (0, 0)),
            pl.BlockSpec((1, _D), lambda i: (0, 0)),
            pl.BlockSpec((_D, _NCLS), lambda i: (0, 0)),
            pl.BlockSpec((1, _NCLS), lambda i: (0, 0)),
        ],
        out_specs=pl.BlockSpec((_NG, _NCLS), lambda i: (0, 0)),
        out_shape=jax.ShapeDtypeStruct((_NG, _NCLS), jnp.float32),
        scratch_shapes=[
            pltpu.VMEM((_NG, _D), jnp.float32),
            pltpu.VMEM((_NG, 1), jnp.float32),
        ],
    )(xl, batch3, lin1_W, lin1_b, lin2_W, lin2_b)


def kernel(x, edge_index, batch,
           conv0_W1, conv0_b1, conv0_W2, conv0_b2,
           conv1_W1, conv1_b1, conv1_W2, conv1_b2,
           conv2_W1, conv2_b1, conv2_W2, conv2_b2,
           lin1_W, lin1_b, lin2_W, lin2_b):
    src = edge_index[0]
    dst = edge_index[1]
    zfull = jnp.zeros((_N, _D), jnp.float32)

    h = x
    for (W1, b1, W2, b2) in (
        (conv0_W1, conv0_b1, conv0_W2, conv0_b2),
        (conv1_W1, conv1_b1, conv1_W2, conv1_b2),
    ):
        partial = _sc_agg(h, src, dst, zfull)
        h = _mlp(partial, W1, b1.reshape(1, _D), W2, b2.reshape(1, _D))

    partial = _sc_agg(h, src, dst, zfull)
    h, logsoft = _mlp3_head(partial, conv2_W1, conv2_b1.reshape(1, _D),
                            conv2_W2, conv2_b2.reshape(1, _D),
                            batch, lin1_W, lin1_b.reshape(1, _D),
                            lin2_W, lin2_b.reshape(1, _NCLS))
    return (logsoft, h)
